# baseline (speedup 1.0000x reference)
import jax
import jax.numpy as jnp
from jax import lax
from jax.experimental import pallas as pl
from jax.experimental.pallas import tpu as pltpu
from jax.experimental.pallas import tpu_sc as plsc

N_NODES = 10000
N_EDGES = 320000
D_FEAT = 128

L = 16
NT = 16
NP = 10240
CPT = NP // NT
VPT = CPT // L
EPT = N_EDGES // NT
EVPT = EPT // L


def _mm_body(x_ref, w_ref, o_ref):
    o_ref[...] = jnp.zeros((8, NP), jnp.float32)
    o_ref[:, pl.ds(0, N_NODES)] = lax.dot_general(
        w_ref[...], x_ref[...],
        dimension_numbers=(((0,), (1,)), ((), ())),
        preferred_element_type=jnp.float32)


def _project(xp, w2):
    return pl.pallas_call(
        _mm_body,
        out_shape=jax.ShapeDtypeStruct((8, NP), jnp.float32),
    )(xp, w2)


def _sc_body(edge_hbm, vs_hbm, scal_hbm,
             x1_hbm, x2_hbm, x3_hbm, hg_hbm,
             srcv, dstv, val, acc, rowidx, zrow, redsm, xnb0, xnb1,
             s0v, scalv, hgbuf, hgred, sh_acc, sh_x, sh_hg,
             sem_in, sem_val, sem_o0, sem_o1):
    t = lax.axis_index("s")
    zero16 = jnp.zeros((L,), jnp.float32)
    ROWS = NP // 128
    RPTL = ROWS // NT

    stage = [
        pltpu.async_copy(edge_hbm.at[pl.ds(t * EPT, EPT)], srcv, sem_in),
        pltpu.async_copy(edge_hbm.at[pl.ds(N_EDGES + t * EPT, EPT)],
                         dstv, sem_in),
        pltpu.async_copy(vs_hbm.at[pl.ds(0, NP)], val, sem_in),
        pltpu.async_copy(vs_hbm.at[pl.ds(NP + t * CPT, CPT)], s0v, sem_in),
        pltpu.async_copy(scal_hbm, scalv, sem_in),
    ]

    def pro(j, _):
        rowidx[pl.ds(j * L, L)] = (
            lax.broadcasted_iota(jnp.int32, (L,), 0) + j * L)
        return 0
    lax.fori_loop(0, ROWS // L, pro, 0)
    for r in range(RPTL):
        for u in range(8):
            zrow[r, pl.ds(u * L, L)] = zero16
    pltpu.sync_copy(zrow, sh_acc.at[pl.ds(t * RPTL, RPTL)])

    def zero_acc():
        def zer(r, _):
            for u in range(8):
                acc[r, pl.ds(u * L, L)] = zero16
            return 0
        lax.fori_loop(0, ROWS, zer, 0)

    def edge_pass():
        @plsc.parallel_loop(0, EVPT, unroll=8)
        def _(i):
            o = i * L
            s = srcv[pl.ds(o, L)]
            d = dstv[pl.ds(o, L)]
            v = plsc.load_gather(val, [s])
            plsc.addupdate_scatter(
                acc, [lax.shift_right_logical(d, 7), d & 127], v)

    def combine(layer, xnbuf):
        pltpu.sync_copy(acc, sh_acc.at[rowidx], add=True)
        if layer < 2:
            zero_acc()
        plsc.subcore_barrier()
        pltpu.sync_copy(sh_acc.at[pl.ds(t * RPTL, RPTL)], redsm)
        pltpu.sync_copy(zrow, sh_acc.at[pl.ds(t * RPTL, RPTL)])

        wn = scalv[pl.ds(3 * layer * L, L)]
        bn = scalv[pl.ds((3 * layer + 1) * L, L)]
        ws = scalv[pl.ds((3 * layer + 2) * L, L)]

        @plsc.parallel_loop(0, VPT, unroll=4, carry=zero16)
        def hvec(c, hsum):
            aggv = redsm[lax.shift_right_logical(c, 3),
                         pl.ds((c & 7) * L, L)]
            if layer == 0:
                xn = aggv + bn + s0v[pl.ds(c * L, L)]
            else:
                xn = wn * aggv + bn + ws * val[pl.ds(t * CPT + c * L, L)]
            xn = jnp.maximum(xn, 0.0)
            gid = lax.broadcasted_iota(jnp.int32, (L,), 0) + t * CPT + c * L
            xn = jnp.where(gid < N_NODES, xn, 0.0)
            xnbuf[pl.ds(c * L, L)] = xn
            return hsum + xn
        return hvec

    plsc.subcore_barrier()

    zero_acc()
    for cp in stage:
        cp.wait()
    edge_pass()
    combine(0, xnb0)
    out0 = pltpu.async_copy(xnb0, x1_hbm.at[pl.ds(t * CPT, CPT)], sem_o0)
    pltpu.sync_copy(xnb0, sh_x.at[pl.ds(t * CPT, CPT)])
    plsc.subcore_barrier()
    cpv = pltpu.async_copy(sh_x, val, sem_val)

    cpv.wait()
    edge_pass()
    combine(1, xnb1)
    out1 = pltpu.async_copy(xnb1, x2_hbm.at[pl.ds(t * CPT, CPT)], sem_o1)
    pltpu.sync_copy(xnb1, sh_x.at[pl.ds(t * CPT, CPT)])
    plsc.subcore_barrier()
    cpv = pltpu.async_copy(sh_x, val, sem_val)

    cpv.wait()
    edge_pass()
    out0.wait()
    hvec = combine(2, xnb0)
    out2 = pltpu.async_copy(xnb0, x3_hbm.at[pl.ds(t * CPT, CPT)], sem_o0)

    def zer8(r, _):
        hgbuf[pl.ds(r * L, L)] = zero16
        return 0
    lax.fori_loop(1, 8, zer8, 0)
    hgbuf[pl.ds(0, L)] = hvec
    pltpu.sync_copy(hgbuf, sh_hg.at[pl.ds(t * 128, 128)])
    plsc.subcore_barrier()

    @pl.when(t == 0)
    def _():
        pltpu.sync_copy(sh_hg, hgred)

        def srow(sid, v):
            return v + hgred[pl.ds(sid * 128, L)]
        tot = lax.fori_loop(0, NT, srow, zero16)
        hgbuf[pl.ds(0, L)] = jnp.broadcast_to(jnp.sum(tot), (L,))
        pltpu.sync_copy(hgbuf, hg_hbm)

    out1.wait()
    out2.wait()


_sc_net_cache = []


def _sc_net(*args):
    if not _sc_net_cache:
        _sc_net_cache.append(_make_sc_net())
    return _sc_net_cache[0](*args)


def _make_sc_net():
    return pl.kernel(
        _sc_body,
        out_type=[
        jax.ShapeDtypeStruct((NP,), jnp.float32),
        jax.ShapeDtypeStruct((NP,), jnp.float32),
        jax.ShapeDtypeStruct((NP,), jnp.float32),
        jax.ShapeDtypeStruct((128,), jnp.float32),
    ],
    mesh=plsc.VectorSubcoreMesh(
        core_axis_name="c", subcore_axis_name="s",
        num_cores=1, num_subcores=16),
    compiler_params=pltpu.CompilerParams(needs_layout_passes=False),
    scratch_types=[
        pltpu.VMEM((EPT,), jnp.int32),
        pltpu.VMEM((EPT,), jnp.int32),
        pltpu.VMEM((NP,), jnp.float32),
        pltpu.VMEM((NP // 128, 128), jnp.float32),
        pltpu.VMEM((NP // 128,), jnp.int32),
        pltpu.VMEM((NP // 128 // NT, 128), jnp.float32),
        pltpu.VMEM((NP // 128 // NT, 128), jnp.float32),
        pltpu.VMEM((CPT,), jnp.float32),
        pltpu.VMEM((CPT,), jnp.float32),
        pltpu.VMEM((CPT,), jnp.float32),
        pltpu.VMEM((9 * L,), jnp.float32),
        pltpu.VMEM((128,), jnp.float32),
        pltpu.VMEM((NT * 128,), jnp.float32),
        pltpu.VMEM_SHARED((NP // 128, 128), jnp.float32),
        pltpu.VMEM_SHARED((NP,), jnp.float32),
        pltpu.VMEM_SHARED((NT * 128,), jnp.float32),
        pltpu.SemaphoreType.DMA,
        pltpu.SemaphoreType.DMA,
        pltpu.SemaphoreType.DMA,
        pltpu.SemaphoreType.DMA,
    ],
)


def kernel(x, edge_index, Wn0, bn0, Ws0, Wn1, bn1, Ws1, Wn2, bn2, Ws2):
    edges = edge_index.astype(jnp.int32).reshape(2 * N_EDGES)

    w2 = jnp.concatenate(
        [Wn0, Ws0, jnp.zeros((D_FEAT, 6), jnp.float32)], axis=1)
    vs = _project(x, w2).reshape(8 * NP)

    scal = jnp.concatenate([
        jnp.broadcast_to(bn0[0], (L,)),
        jnp.broadcast_to(bn0[0], (L,)),
        jnp.broadcast_to(bn0[0], (L,)),
        jnp.broadcast_to(Wn1[0, 0], (L,)),
        jnp.broadcast_to(bn1[0], (L,)),
        jnp.broadcast_to(Ws1[0, 0], (L,)),
        jnp.broadcast_to(Wn2[0, 0], (L,)),
        jnp.broadcast_to(bn2[0], (L,)),
        jnp.broadcast_to(Ws2[0, 0], (L,)),
    ])

    x1o, x2o, x3o, hgo = _sc_net(edges, vs, scal)

    x1 = x1o[:N_NODES, None]
    x2 = x2o[:N_NODES, None]
    x3 = x3o[:N_NODES, None]
    hg = hgo[0:1, None]
    return (hg, x, x1, x2, x3)
